# baseline (speedup 1.0000x reference)
import functools

import jax
import jax.numpy as jnp
from jax import lax
from jax.experimental import pallas as pl
from jax.experimental.pallas import tpu as pltpu
from jax.experimental.pallas import tpu_sc as plsc

D_MODEL = 64
CHUNK = 512


@jax.jit
def _embedding_lookup(idx_flat, table):
    B = idx_flat.shape[0]
    info = plsc.get_sparse_core_info()
    nw = info.num_cores * info.num_subcores
    b_per_w = B // nw
    n_chunks = b_per_w // CHUNK
    assert b_per_w * nw == B and n_chunks * CHUNK == b_per_w

    mesh = plsc.VectorSubcoreMesh(core_axis_name="c", subcore_axis_name="s")

    @functools.partial(
        pl.kernel,
        mesh=mesh,
        out_type=jax.ShapeDtypeStruct((B, D_MODEL), jnp.float32),
        scratch_types=[
            pltpu.VMEM((CHUNK,), jnp.int32),
            pltpu.VMEM((CHUNK, D_MODEL), jnp.float32),
            pltpu.SemaphoreType.DMA,
        ],
        compiler_params=pltpu.CompilerParams(use_tc_tiling_on_sc=False),
    )
    def k(table_hbm, idx_hbm, out_hbm, idx_v, rows_v, sem):
        wid = lax.axis_index("s") * info.num_cores + lax.axis_index("c")
        base = wid * b_per_w

        def body(g, carry):
            off = base + g * CHUNK
            pltpu.sync_copy(idx_hbm.at[pl.ds(off, CHUNK)], idx_v)
            pltpu.async_copy(table_hbm.at[idx_v], rows_v, sem).wait()
            pltpu.sync_copy(rows_v, out_hbm.at[pl.ds(off, CHUNK)])
            return carry

        lax.fori_loop(0, n_chunks, body, 0)

    return k(table, idx_flat)


def kernel(x, table):
    out = _embedding_lookup(x.reshape(-1), table)
    return out.reshape(x.shape + (D_MODEL,))


# speedup vs baseline: 1.0480x; 1.0480x over previous
import functools

import jax
import jax.numpy as jnp
from jax import lax
from jax.experimental import pallas as pl
from jax.experimental.pallas import tpu as pltpu
from jax.experimental.pallas import tpu_sc as plsc

D_MODEL = 64
CHUNK = 512


@jax.jit
def _embedding_lookup(idx, table):
    nw_in, n_chunks, _ = idx.shape
    B = idx.size
    info = plsc.get_sparse_core_info()
    nw = info.num_cores * info.num_subcores
    assert nw_in == nw and n_chunks % 2 == 0
    b_per_w = B // nw
    n_pairs = n_chunks // 2

    mesh = plsc.VectorSubcoreMesh(core_axis_name="c", subcore_axis_name="s")

    @functools.partial(
        pl.kernel,
        mesh=mesh,
        out_type=jax.ShapeDtypeStruct((B, D_MODEL), jnp.float32),
        scratch_types=[
            pltpu.VMEM((n_chunks, CHUNK), jnp.int32),
            pltpu.VMEM((CHUNK, D_MODEL), jnp.float32),
            pltpu.VMEM((CHUNK, D_MODEL), jnp.float32),
            pltpu.SemaphoreType.DMA,
            pltpu.SemaphoreType.DMA,
            pltpu.SemaphoreType.DMA,
            pltpu.SemaphoreType.DMA,
        ],
        compiler_params=pltpu.CompilerParams(use_tc_tiling_on_sc=False),
    )
    def k(table_hbm, idx_hbm, out_hbm, idx_v, rows0, rows1, g0s, g1s, o0s, o1s):
        wid = lax.axis_index("s") * info.num_cores + lax.axis_index("c")
        base = wid * b_per_w

        def gat(g, rows, sem):
            return pltpu.make_async_copy(table_hbm.at[idx_v.at[g]], rows, sem)

        def put(g, rows, sem):
            return pltpu.make_async_copy(
                rows, out_hbm.at[pl.ds(base + g * CHUNK, CHUNK)], sem)

        pltpu.sync_copy(idx_hbm.at[wid], idx_v)
        gat(0, rows0, g0s).start()

        def body(j, carry):
            g0 = 2 * j
            g1 = g0 + 1

            @pl.when(j > 0)
            def _():
                put(g0 - 1, rows1, o1s).wait()

            gat(g1, rows1, g1s).start()
            gat(g0, rows0, g0s).wait()
            put(g0, rows0, o0s).start()

            @pl.when(j < n_pairs - 1)
            def _():
                put(g0, rows0, o0s).wait()
                gat(g0 + 2, rows0, g0s).start()

            gat(g1, rows1, g1s).wait()
            put(g1, rows1, o1s).start()
            return carry

        lax.fori_loop(0, n_pairs, body, 0)
        put(n_chunks - 2, rows0, o0s).wait()
        put(n_chunks - 1, rows1, o1s).wait()

    return k(table, idx)


def kernel(x, table):
    info = plsc.get_sparse_core_info()
    nw = info.num_cores * info.num_subcores
    idx = x.reshape(nw, -1, CHUNK)
    out = _embedding_lookup(idx, table)
    return out.reshape(x.shape + (D_MODEL,))
